# baseline (speedup 1.0000x reference)
import functools

import jax
import jax.numpy as jnp
from jax import lax
from jax.experimental import pallas as pl
from jax.experimental.pallas import tpu as pltpu
from jax.experimental.pallas import tpu_sc as plsc

N_NODES = 10000
N_EDGES = 320000
D_FEAT = 128
D_HID1 = 50
D_HID2 = 128
N_CLASSES = 40

HP = 64
CPS = HP // 2
NP = 10240
NC, NS = 2, 16
NTILES = NC * NS
RPT = NP // NS
HB = RPT // 2
CHUNK = 128
NROWS = 2560
EP = NROWS * CHUNK
NCH_DEG = NROWS // NTILES
NCH_AGG = NROWS // NS

_mesh = plsc.VectorSubcoreMesh(core_axis_name="c", subcore_axis_name="s")
_SC_PARAMS = pltpu.CompilerParams(use_tc_tiling_on_sc=False)


@functools.partial(
    pl.kernel,
    out_type=jax.ShapeDtypeStruct((NC * NP, 16), jnp.float32),
    mesh=_mesh,
    compiler_params=_SC_PARAMS,
    scratch_types=[
        pltpu.VMEM((NCH_DEG, CHUNK), jnp.int32),
        pltpu.VMEM((CHUNK, 16), jnp.float32),
        pltpu.VMEM_SHARED((NP, 16), jnp.float32),
        pltpu.SemaphoreType.DMA,
    ],
)
def _sc_degree(dst_hbm, ones_hbm, zeros_hbm, out_hbm, idx_v, ones_v, acc_sh,
               ssem):
    c = lax.axis_index("c")
    s = lax.axis_index("s")
    tid = c * NS + s
    pltpu.sync_copy(zeros_hbm.at[pl.ds(s * RPT, RPT)],
                    acc_sh.at[pl.ds(s * RPT, RPT)])
    pltpu.sync_copy(dst_hbm.at[pl.ds(tid * NCH_DEG, NCH_DEG)], idx_v)
    pltpu.sync_copy(ones_hbm, ones_v)
    plsc.subcore_barrier()

    def body(j, carry):
        pltpu.make_async_copy(ones_v, acc_sh.at[idx_v.at[j]],
                              ssem).start(add=True)

        @pl.when(j >= 8)
        def _():
            pltpu.make_async_copy(ones_v, acc_sh.at[idx_v.at[0]], ssem).wait()

        return carry

    lax.fori_loop(0, NCH_DEG, body, 0)

    def drain(j, carry):
        pltpu.make_async_copy(ones_v, acc_sh.at[idx_v.at[0]], ssem).wait()
        return carry

    lax.fori_loop(0, 8, drain, 0)
    plsc.subcore_barrier()
    pltpu.sync_copy(acc_sh.at[pl.ds(s * RPT, RPT)],
                    out_hbm.at[pl.ds(c * NP + s * RPT, RPT)])


def _agg_loop(y_sh, acc_sh, sidx, didx, rows0, rows1, gs0, gs1, ss0, ss1):
    pltpu.async_copy(y_sh.at[sidx.at[0]], rows0, gs0)
    pltpu.async_copy(y_sh.at[sidx.at[1]], rows1, gs1)

    def body(i, carry):
        j0 = 2 * i
        j1 = j0 + 1
        bufs = ((rows0, gs0, ss0, j0), (rows1, gs1, ss1, j1))
        for rows, gs, ss, j in bufs:
            pltpu.make_async_copy(y_sh.at[sidx.at[j]], rows, gs).wait()
            pltpu.make_async_copy(rows, acc_sh.at[didx.at[j]],
                                  ss).start(add=True)
        for rows, gs, ss, j in bufs:
            @pl.when(j + 2 < NCH_AGG)
            def _():
                pltpu.make_async_copy(rows, acc_sh.at[didx.at[j]], ss).wait()
                pltpu.async_copy(y_sh.at[sidx.at[j + 2]], rows, gs)

        return carry

    lax.fori_loop(0, NCH_AGG // 2, body, 0)
    pltpu.make_async_copy(rows0, acc_sh.at[didx.at[0]], ss0).wait()
    pltpu.make_async_copy(rows1, acc_sh.at[didx.at[0]], ss1).wait()


def _newton_rsqrt(p):
    i = lax.bitcast_convert_type(p, jnp.int32)
    i = jnp.int32(0x5F3759DF) - jnp.right_shift(i, jnp.int32(1))
    y = lax.bitcast_convert_type(i, jnp.float32)
    y = y * (1.5 - 0.5 * p * y * y)
    y = y * (1.5 - 0.5 * p * y * y)
    y = y * (1.5 - 0.5 * p * y * y)
    return y


@functools.partial(
    pl.kernel,
    out_type=jax.ShapeDtypeStruct((NP, 128), jnp.float32),
    mesh=_mesh,
    compiler_params=_SC_PARAMS,
    scratch_types=[
        pltpu.VMEM((NCH_AGG, CHUNK), jnp.int32),
        pltpu.VMEM((NCH_AGG, CHUNK), jnp.int32),
        pltpu.VMEM((CHUNK, CPS), jnp.float32),
        pltpu.VMEM((CHUNK, CPS), jnp.float32),
        pltpu.VMEM((HB, CPS), jnp.float32),
        pltpu.VMEM((HB, CPS), jnp.float32),
        pltpu.VMEM((RPT, 16), jnp.float32),
        pltpu.VMEM((HB, 16), jnp.float32),
        pltpu.VMEM((8, CPS), jnp.float32),
        pltpu.VMEM_SHARED((NP, CPS), jnp.float32),
        pltpu.VMEM_SHARED((NP, CPS), jnp.float32),
        pltpu.SemaphoreType.DMA,
        pltpu.SemaphoreType.DMA,
        pltpu.SemaphoreType.DMA,
        pltpu.SemaphoreType.DMA,
    ],
)
def _sc_fused(xw_hbm, src_hbm, dst_hbm, deg_hbm, b1s_hbm, zeros_hbm, a2s_hbm,
              sidx, didx, rows0, rows1, accv, yv, disfull, degb, b1v,
              acc_sh, y_sh, gs0, gs1, ss0, ss1):
    c = lax.axis_index("c")
    s = lax.axis_index("s")
    r0 = s * RPT
    pltpu.sync_copy(zeros_hbm.at[pl.ds(r0, RPT)], acc_sh.at[pl.ds(r0, RPT)])
    pltpu.sync_copy(src_hbm.at[pl.ds(s * NCH_AGG, NCH_AGG)], sidx)
    pltpu.sync_copy(dst_hbm.at[pl.ds(s * NCH_AGG, NCH_AGG)], didx)
    pltpu.sync_copy(b1s_hbm.at[pl.ds(0, 8), pl.ds(c * CPS, CPS)], b1v)

    for h in range(2):
        rh = r0 + h * HB
        pltpu.sync_copy(deg_hbm.at[pl.ds(rh, HB)],
                        disfull.at[pl.ds(h * HB, HB)])
        pltpu.sync_copy(deg_hbm.at[pl.ds(NP + rh, HB)], degb)
        pltpu.sync_copy(xw_hbm.at[pl.ds(rh, HB), pl.ds(c * CPS, CPS)], accv)

        def prebody(r4, carry):
            for k in range(4):
                r = 4 * r4 + k
                p = disfull[h * HB + r, :] + degb[r, :] + 1.0
                y = _newton_rsqrt(p)
                disfull[h * HB + r, :] = y
                accv[r, 0:16] = accv[r, 0:16] * y
                accv[r, 16:32] = accv[r, 16:32] * y
            return carry

        lax.fori_loop(0, HB // 4, prebody, 0)
        pltpu.sync_copy(accv, y_sh.at[pl.ds(rh, HB)])
    plsc.subcore_barrier()

    _agg_loop(y_sh, acc_sh, sidx, didx, rows0, rows1, gs0, gs1, ss0, ss1)
    plsc.subcore_barrier()

    b1a = b1v[0, 0:16]
    b1b = b1v[0, 16:32]
    for h in range(2):
        rh = r0 + h * HB
        pltpu.sync_copy(acc_sh.at[pl.ds(rh, HB)], accv)
        pltpu.sync_copy(y_sh.at[pl.ds(rh, HB)], yv)

        def midbody(r4, carry):
            for k in range(4):
                r = 4 * r4 + k
                dv = disfull[h * HB + r, :]
                o0 = jnp.maximum((accv[r, 0:16] + yv[r, 0:16]) * dv + b1a, 0.0)
                o1 = jnp.maximum((accv[r, 16:32] + yv[r, 16:32]) * dv + b1b, 0.0)
                yv[r, 0:16] = o0 * dv
                yv[r, 16:32] = o1 * dv
            return carry

        lax.fori_loop(0, HB // 4, midbody, 0)
        pltpu.sync_copy(yv, y_sh.at[pl.ds(rh, HB)])
        pltpu.sync_copy(zeros_hbm.at[pl.ds(rh, HB)], acc_sh.at[pl.ds(rh, HB)])
    plsc.subcore_barrier()

    _agg_loop(y_sh, acc_sh, sidx, didx, rows0, rows1, gs0, gs1, ss0, ss1)
    plsc.subcore_barrier()

    for h in range(2):
        rh = r0 + h * HB
        pltpu.sync_copy(acc_sh.at[pl.ds(rh, HB)], accv)
        pltpu.sync_copy(y_sh.at[pl.ds(rh, HB)], yv)

        def finbody(r4, carry):
            for k in range(4):
                r = 4 * r4 + k
                dv = disfull[h * HB + r, :]
                accv[r, 0:16] = (accv[r, 0:16] + yv[r, 0:16]) * dv
                accv[r, 16:32] = (accv[r, 16:32] + yv[r, 16:32]) * dv
            return carry

        lax.fori_loop(0, HB // 4, finbody, 0)
        pltpu.sync_copy(accv, a2s_hbm.at[pl.ds(rh, HB), pl.ds(c * CPS, CPS)])


_TC_PARAMS = pltpu.CompilerParams(vmem_limit_bytes=100 * 1024 * 1024)
_GBLK = 2000


def _tc_xw_body(x_ref, w_ref, y_ref):
    y_ref[...] = jnp.dot(x_ref[...], w_ref[...],
                         preferred_element_type=jnp.float32)


_tc_xw = pl.pallas_call(
    _tc_xw_body,
    grid=(N_NODES // _GBLK,),
    in_specs=[pl.BlockSpec((_GBLK, D_FEAT), lambda i: (i, 0)),
              pl.BlockSpec((D_FEAT, 128), lambda i: (0, 0))],
    out_specs=pl.BlockSpec((_GBLK, 128), lambda i: (i, 0)),
    out_shape=jax.ShapeDtypeStruct((NP, 128), jnp.float32),
    compiler_params=_TC_PARAMS,
)


def _tc_final_body(a_ref, w2_ref, b2_ref, wl_ref, bl_ref, out_ref, h_ref):
    a2 = a_ref[:, 0:HP]
    h = jnp.dot(a2, w2_ref[...], preferred_element_type=jnp.float32) + b2_ref[...]
    h_ref[...] = h
    out = jnp.dot(h, wl_ref[...], preferred_element_type=jnp.float32) + bl_ref[...]
    out_ref[...] = out[:, 0:N_CLASSES]


_tc_final = pl.pallas_call(
    _tc_final_body,
    grid=(N_NODES // _GBLK,),
    in_specs=[pl.BlockSpec((_GBLK, 128), lambda i: (i, 0)),
              pl.BlockSpec((HP, D_HID2), lambda i: (0, 0)),
              pl.BlockSpec((1, D_HID2), lambda i: (0, 0)),
              pl.BlockSpec((D_HID2, 128), lambda i: (0, 0)),
              pl.BlockSpec((1, 128), lambda i: (0, 0))],
    out_specs=(pl.BlockSpec((_GBLK, N_CLASSES), lambda i: (i, 0)),
               pl.BlockSpec((_GBLK, D_HID2), lambda i: (i, 0))),
    out_shape=(jax.ShapeDtypeStruct((N_NODES, N_CLASSES), jnp.float32),
               jax.ShapeDtypeStruct((N_NODES, D_HID2), jnp.float32)),
    compiler_params=_TC_PARAMS,
)


def kernel(x, edge_index, W1, b1, W2, b2, Wl, bl):
    f32 = jnp.float32
    pad_e = EP - N_EDGES
    src_p = jnp.concatenate(
        [edge_index[0], jnp.zeros((pad_e,), jnp.int32)]).reshape(NROWS, CHUNK)
    dst_p = jnp.concatenate(
        [edge_index[1], jnp.full((pad_e,), NP - 1, jnp.int32)]).reshape(NROWS, CHUNK)
    W1pp = jnp.zeros((D_FEAT, 128), f32).at[:, :D_HID1].set(W1)
    b1s = jnp.zeros((8, 128), f32).at[:, :D_HID1].set(
        jnp.broadcast_to(b1, (8, D_HID1)))
    W2p = jnp.zeros((HP, D_HID2), f32).at[:D_HID1].set(W2)
    b2p = b2.reshape(1, D_HID2)
    Wlp = jnp.zeros((D_HID2, 128), f32).at[:, :N_CLASSES].set(Wl)
    blp = jnp.zeros((1, 128), f32).at[0, :N_CLASSES].set(bl)

    ones16 = jnp.ones((CHUNK, 16), f32)
    z16 = jnp.zeros((NP, 16), f32)
    z32 = jnp.zeros((NP, CPS), f32)

    deg2 = _sc_degree(dst_p, ones16, z16)
    xw = _tc_xw(x, W1pp)
    a2s = _sc_fused(xw, src_p, dst_p, deg2, b1s, z32)
    outp, h = _tc_final(a2s, W2p, b2p, Wlp, blp)
    return (outp, h)


# speedup vs baseline: 1.1598x; 1.1598x over previous
import functools

import jax
import jax.numpy as jnp
from jax import lax
from jax.experimental import pallas as pl
from jax.experimental.pallas import tpu as pltpu
from jax.experimental.pallas import tpu_sc as plsc

N_NODES = 10000
N_EDGES = 320000
D_FEAT = 128
D_HID1 = 50
D_HID2 = 128
N_CLASSES = 40

HP = 64
CPS = HP // 2
NP = 10240
NC, NS = 2, 16
NTILES = NC * NS
RPT = NP // NS
HB = RPT // 2
CHUNK = 128
EROWS = N_EDGES // CHUNK
DOFF = EROWS

_mesh = plsc.VectorSubcoreMesh(core_axis_name="c", subcore_axis_name="s")
_SC_PARAMS = pltpu.CompilerParams(use_tc_tiling_on_sc=False)


@functools.partial(
    pl.kernel,
    out_type=jax.ShapeDtypeStruct((NC * NP, 16), jnp.float32),
    mesh=_mesh,
    compiler_params=_SC_PARAMS,
    scratch_types=[
        pltpu.VMEM((79, CHUNK), jnp.int32),
        pltpu.VMEM((CHUNK, 16), jnp.float32),
        pltpu.VMEM_SHARED((NP, 16), jnp.float32),
        pltpu.SemaphoreType.DMA,
    ],
)
def _sc_degree(ei_hbm, ones_hbm, zeros_hbm, out_hbm, idx_v, ones_v, acc_sh,
               ssem):
    c = lax.axis_index("c")
    s = lax.axis_index("s")
    tid = c * NS + s
    start = 78 * tid + jnp.minimum(tid, 4)
    nch = 78 + (tid < 4).astype(jnp.int32)
    pltpu.sync_copy(zeros_hbm.at[pl.ds(s * RPT, RPT)],
                    acc_sh.at[pl.ds(s * RPT, RPT)])
    pltpu.sync_copy(ei_hbm.at[pl.ds(DOFF + start, 78)],
                    idx_v.at[pl.ds(0, 78)])

    @pl.when(tid < 4)
    def _():
        pltpu.sync_copy(ei_hbm.at[pl.ds(DOFF + start + 78, 1)],
                        idx_v.at[pl.ds(78, 1)])

    pltpu.sync_copy(ones_hbm, ones_v)
    plsc.subcore_barrier()

    def body(j, carry):
        pltpu.make_async_copy(ones_v, acc_sh.at[idx_v.at[j]],
                              ssem).start(add=True)

        @pl.when(j >= 8)
        def _():
            pltpu.make_async_copy(ones_v, acc_sh.at[idx_v.at[0]], ssem).wait()

        return carry

    lax.fori_loop(0, nch, body, 0)

    def drain(j, carry):
        pltpu.make_async_copy(ones_v, acc_sh.at[idx_v.at[0]], ssem).wait()
        return carry

    lax.fori_loop(0, 8, drain, 0)
    plsc.subcore_barrier()
    pltpu.sync_copy(acc_sh.at[pl.ds(s * RPT, RPT)],
                    out_hbm.at[pl.ds(c * NP + s * RPT, RPT)])


def _agg_loop(nch, y_sh, acc_sh, sidx, didx, rows0, rows1, sem0, sem1):
    pltpu.async_copy(y_sh.at[sidx.at[0]], rows0, sem0)

    def body(i, carry):
        j0 = 2 * i
        pltpu.async_copy(y_sh.at[sidx.at[j0 + 1]], rows1, sem1)
        pltpu.make_async_copy(y_sh.at[sidx.at[j0]], rows0, sem0).wait()
        pltpu.sync_copy(rows0, acc_sh.at[didx.at[j0]], add=True)

        @pl.when(j0 + 2 < nch)
        def _():
            pltpu.async_copy(y_sh.at[sidx.at[j0 + 2]], rows0, sem0)

        pltpu.make_async_copy(y_sh.at[sidx.at[j0 + 1]], rows1, sem1).wait()
        pltpu.sync_copy(rows1, acc_sh.at[didx.at[j0 + 1]], add=True)
        return carry

    lax.fori_loop(0, 78, body, 0)

    @pl.when(nch > 156)
    def _():
        pltpu.make_async_copy(y_sh.at[sidx.at[156]], rows0, sem0).wait()
        pltpu.sync_copy(rows0, acc_sh.at[didx.at[156]], add=True)


def _newton_rsqrt(p):
    i = lax.bitcast_convert_type(p, jnp.int32)
    i = jnp.int32(0x5F3759DF) - jnp.right_shift(i, jnp.int32(1))
    y = lax.bitcast_convert_type(i, jnp.float32)
    y = y * (1.5 - 0.5 * p * y * y)
    y = y * (1.5 - 0.5 * p * y * y)
    y = y * (1.5 - 0.5 * p * y * y)
    return y


@functools.partial(
    pl.kernel,
    out_type=jax.ShapeDtypeStruct((NP, 128), jnp.float32),
    mesh=_mesh,
    compiler_params=_SC_PARAMS,
    scratch_types=[
        pltpu.VMEM((157, CHUNK), jnp.int32),
        pltpu.VMEM((157, CHUNK), jnp.int32),
        pltpu.VMEM((CHUNK, CPS), jnp.float32),
        pltpu.VMEM((CHUNK, CPS), jnp.float32),
        pltpu.VMEM((HB, CPS), jnp.float32),
        pltpu.VMEM((HB, CPS), jnp.float32),
        pltpu.VMEM((RPT, 16), jnp.float32),
        pltpu.VMEM((HB, 16), jnp.float32),
        pltpu.VMEM((8, CPS), jnp.float32),
        pltpu.VMEM_SHARED((NP, CPS), jnp.float32),
        pltpu.VMEM_SHARED((NP, CPS), jnp.float32),
        pltpu.SemaphoreType.DMA,
        pltpu.SemaphoreType.DMA,
    ],
)
def _sc_fused(xw_hbm, ei_hbm, deg_hbm, b1s_hbm, zeros_hbm, a2s_hbm,
              sidx, didx, rows0, rows1, accv, yv, disfull, degb, b1v,
              acc_sh, y_sh, sem0, sem1):
    c = lax.axis_index("c")
    s = lax.axis_index("s")
    r0 = s * RPT
    start = 156 * s + jnp.minimum(s, 4)
    nch = 156 + (s < 4).astype(jnp.int32)
    pltpu.sync_copy(zeros_hbm.at[pl.ds(r0, RPT)], acc_sh.at[pl.ds(r0, RPT)])
    pltpu.sync_copy(ei_hbm.at[pl.ds(start, 156)], sidx.at[pl.ds(0, 156)])
    pltpu.sync_copy(ei_hbm.at[pl.ds(DOFF + start, 156)],
                    didx.at[pl.ds(0, 156)])

    @pl.when(s < 4)
    def _():
        pltpu.sync_copy(ei_hbm.at[pl.ds(start + 156, 1)],
                        sidx.at[pl.ds(156, 1)])
        pltpu.sync_copy(ei_hbm.at[pl.ds(DOFF + start + 156, 1)],
                        didx.at[pl.ds(156, 1)])

    pltpu.sync_copy(b1s_hbm.at[pl.ds(0, 8), pl.ds(c * CPS, CPS)], b1v)

    for h in range(2):
        rh = r0 + h * HB
        pltpu.sync_copy(deg_hbm.at[pl.ds(rh, HB)],
                        disfull.at[pl.ds(h * HB, HB)])
        pltpu.sync_copy(deg_hbm.at[pl.ds(NP + rh, HB)], degb)
        pltpu.sync_copy(xw_hbm.at[pl.ds(rh, HB), pl.ds(c * CPS, CPS)], accv)

        def prebody(r4, carry):
            for k in range(4):
                r = 4 * r4 + k
                p = disfull[h * HB + r, :] + degb[r, :] + 1.0
                y = _newton_rsqrt(p)
                disfull[h * HB + r, :] = y
                accv[r, 0:16] = accv[r, 0:16] * y
                accv[r, 16:32] = accv[r, 16:32] * y
            return carry

        lax.fori_loop(0, HB // 4, prebody, 0)
        pltpu.sync_copy(accv, y_sh.at[pl.ds(rh, HB)])
    plsc.subcore_barrier()

    _agg_loop(nch, y_sh, acc_sh, sidx, didx, rows0, rows1, sem0, sem1)
    plsc.subcore_barrier()

    b1a = b1v[0, 0:16]
    b1b = b1v[0, 16:32]
    for h in range(2):
        rh = r0 + h * HB
        pltpu.sync_copy(acc_sh.at[pl.ds(rh, HB)], accv)
        pltpu.sync_copy(y_sh.at[pl.ds(rh, HB)], yv)

        def midbody(r4, carry):
            for k in range(4):
                r = 4 * r4 + k
                dv = disfull[h * HB + r, :]
                o0 = jnp.maximum((accv[r, 0:16] + yv[r, 0:16]) * dv + b1a, 0.0)
                o1 = jnp.maximum((accv[r, 16:32] + yv[r, 16:32]) * dv + b1b, 0.0)
                yv[r, 0:16] = o0 * dv
                yv[r, 16:32] = o1 * dv
            return carry

        lax.fori_loop(0, HB // 4, midbody, 0)
        pltpu.sync_copy(yv, y_sh.at[pl.ds(rh, HB)])
        pltpu.sync_copy(zeros_hbm.at[pl.ds(rh, HB)], acc_sh.at[pl.ds(rh, HB)])
    plsc.subcore_barrier()

    _agg_loop(nch, y_sh, acc_sh, sidx, didx, rows0, rows1, sem0, sem1)
    plsc.subcore_barrier()

    for h in range(2):
        rh = r0 + h * HB
        pltpu.sync_copy(acc_sh.at[pl.ds(rh, HB)], accv)
        pltpu.sync_copy(y_sh.at[pl.ds(rh, HB)], yv)

        def finbody(r4, carry):
            for k in range(4):
                r = 4 * r4 + k
                dv = disfull[h * HB + r, :]
                accv[r, 0:16] = (accv[r, 0:16] + yv[r, 0:16]) * dv
                accv[r, 16:32] = (accv[r, 16:32] + yv[r, 16:32]) * dv
            return carry

        lax.fori_loop(0, HB // 4, finbody, 0)
        pltpu.sync_copy(accv, a2s_hbm.at[pl.ds(rh, HB), pl.ds(c * CPS, CPS)])


_TC_PARAMS = pltpu.CompilerParams(vmem_limit_bytes=100 * 1024 * 1024)
_GBLK = 2000


def _tc_xw_body(x_ref, w_ref, y_ref):
    y_ref[...] = jnp.dot(x_ref[...], w_ref[...],
                         preferred_element_type=jnp.float32)


_tc_xw = pl.pallas_call(
    _tc_xw_body,
    grid=(N_NODES // _GBLK,),
    in_specs=[pl.BlockSpec((_GBLK, D_FEAT), lambda i: (i, 0)),
              pl.BlockSpec((D_FEAT, 128), lambda i: (0, 0))],
    out_specs=pl.BlockSpec((_GBLK, 128), lambda i: (i, 0)),
    out_shape=jax.ShapeDtypeStruct((NP, 128), jnp.float32),
    compiler_params=_TC_PARAMS,
)


def _tc_final_body(a_ref, w2_ref, b2_ref, wl_ref, bl_ref, out_ref, h_ref):
    a2 = a_ref[:, 0:HP]
    h = jnp.dot(a2, w2_ref[...], preferred_element_type=jnp.float32) + b2_ref[...]
    h_ref[...] = h
    out = jnp.dot(h, wl_ref[...], preferred_element_type=jnp.float32) + bl_ref[...]
    out_ref[...] = out[:, 0:N_CLASSES]


_tc_final = pl.pallas_call(
    _tc_final_body,
    grid=(N_NODES // _GBLK,),
    in_specs=[pl.BlockSpec((_GBLK, 128), lambda i: (i, 0)),
              pl.BlockSpec((HP, D_HID2), lambda i: (0, 0)),
              pl.BlockSpec((1, D_HID2), lambda i: (0, 0)),
              pl.BlockSpec((D_HID2, 128), lambda i: (0, 0)),
              pl.BlockSpec((1, 128), lambda i: (0, 0))],
    out_specs=(pl.BlockSpec((_GBLK, N_CLASSES), lambda i: (i, 0)),
               pl.BlockSpec((_GBLK, D_HID2), lambda i: (i, 0))),
    out_shape=(jax.ShapeDtypeStruct((N_NODES, N_CLASSES), jnp.float32),
               jax.ShapeDtypeStruct((N_NODES, D_HID2), jnp.float32)),
    compiler_params=_TC_PARAMS,
)


def kernel(x, edge_index, W1, b1, W2, b2, Wl, bl):
    f32 = jnp.float32
    ei2d = edge_index.reshape(2 * EROWS, CHUNK)
    W1pp = jnp.zeros((D_FEAT, 128), f32).at[:, :D_HID1].set(W1)
    b1s = jnp.zeros((8, 128), f32).at[:, :D_HID1].set(
        jnp.broadcast_to(b1, (8, D_HID1)))
    W2p = jnp.zeros((HP, D_HID2), f32).at[:D_HID1].set(W2)
    b2p = b2.reshape(1, D_HID2)
    Wlp = jnp.zeros((D_HID2, 128), f32).at[:, :N_CLASSES].set(Wl)
    blp = jnp.zeros((1, 128), f32).at[0, :N_CLASSES].set(bl)

    ones16 = jnp.ones((CHUNK, 16), f32)
    z16 = jnp.zeros((NP, 16), f32)
    z32 = jnp.zeros((NP, CPS), f32)

    deg2 = _sc_degree(ei2d, ones16, z16)
    xw = _tc_xw(x, W1pp)
    a2s = _sc_fused(xw, ei2d, deg2, b1s, z32)
    outp, h = _tc_final(a2s, W2p, b2p, Wlp, blp)
    return (outp, h)
